# baseline (speedup 1.0000x reference)
import functools

import jax
import jax.numpy as jnp
import numpy as np
from jax import lax
from jax.experimental import pallas as pl
from jax.experimental.pallas import tpu as pltpu
from jax.experimental.pallas import tpu_sc as plsc

_N = 10000
_E = 320000
_H = 128
_G = 128
_C = 10
_NC = 2
_NS = 16
_NW = _NC * _NS
_CH = 88
_NCHUNK = 120
_CPP = 24
_NRING = 3
_EPW = _NCHUNK * _CH
_EP = _NW * _EPW
_NP = 10240
_RPT = _NP // _NS
_BN = float(1.0 / np.sqrt(1.0 + 1e-5))
_BLK = 2000
_NBLK = _N // _BLK


def _sc_segment_sum(h, src, dst, zrows):
    mesh = plsc.VectorSubcoreMesh(
        core_axis_name="c", subcore_axis_name="s",
        num_cores=_NC, num_subcores=_NS)

    @functools.partial(
        pl.kernel,
        out_type=jax.ShapeDtypeStruct((_NC, _NP, _H), jnp.float32),
        mesh=mesh,
        scratch_types=[
            pltpu.VMEM((_CPP, _CH), jnp.int32),
            pltpu.VMEM((_CPP, _CH), jnp.int32),
            [pltpu.VMEM((_CH, _H), jnp.float32) for _ in range(_NRING)],
            pltpu.VMEM_SHARED((_NP, _H), jnp.float32),
            [pltpu.SemaphoreType.DMA for _ in range(_NRING)],
        ],
    )
    def seg_sum(h_hbm, src_hbm, dst_hbm, z_hbm, out_hbm,
                src_v, dst_v, rows, acc_sh, sems):
        c = lax.axis_index("c")
        s = lax.axis_index("s")
        wid = s * _NC + c
        rbase = s * _RPT
        pltpu.sync_copy(z_hbm.at[pl.ds(rbase, _RPT)],
                        acc_sh.at[pl.ds(rbase, _RPT)])
        plsc.subcore_barrier()

        def phase(p, carry):
            pltpu.sync_copy(src_hbm.at[wid, p], src_v)
            pltpu.sync_copy(dst_hbm.at[wid, p], dst_v)

            for b in range(_NRING):
                pltpu.async_copy(h_hbm.at[src_v.at[b]], rows[b], sems[b])

            def body(j, carry2):
                for b in range(_NRING):
                    ch = _NRING * j + b
                    pltpu.make_async_copy(h_hbm.at[src_v.at[ch]], rows[b],
                                          sems[b]).wait()
                    pltpu.sync_copy(rows[b], acc_sh.at[dst_v.at[ch]],
                                    add=True)

                    @pl.when(ch + _NRING < _CPP)
                    def _():
                        pltpu.async_copy(h_hbm.at[src_v.at[ch + _NRING]],
                                         rows[b], sems[b])
                return carry2

            lax.fori_loop(0, _CPP // _NRING, body, 0)
            return carry

        lax.fori_loop(0, _NCHUNK // _CPP, phase, 0)
        plsc.subcore_barrier()
        pltpu.sync_copy(acc_sh.at[pl.ds(rbase, _RPT)],
                        out_hbm.at[c, pl.ds(rbase, _RPT)])

    return seg_sum(h, src, dst, zrows)


def _mlp_pool_body(h_ref, acc_ref, b_ref, W1_ref, b1_ref, g_ref, be_ref,
                   W2_ref, b2_ref, hout_ref, pool_ref):
    i = pl.program_id(0)
    hin = h_ref[...] + acc_ref[0] + acc_ref[1]
    a = jnp.dot(hin, W1_ref[...], preferred_element_type=jnp.float32)
    a = (a + b1_ref[...]) * (g_ref[...] * _BN) + be_ref[...]
    a = jnp.maximum(a, 0.0)
    hout = jnp.dot(a, W2_ref[...], preferred_element_type=jnp.float32)
    hout = jnp.maximum(hout + b2_ref[...], 0.0)
    hout_ref[...] = hout
    bvals = b_ref[...][:, 0]
    onehot = (lax.broadcasted_iota(jnp.int32, (_G, _BLK), 0)
              == bvals[None, :]).astype(jnp.float32)
    pp = jnp.dot(onehot, hout, preferred_element_type=jnp.float32)

    @pl.when(i == 0)
    def _():
        pool_ref[...] = jnp.zeros_like(pool_ref)

    pool_ref[...] += pp


def _tc_mlp_pool(h, acc, batch2, W1, b1, g, be, W2, b2):
    full = lambda shape: pl.BlockSpec(shape, lambda i: (0,) * len(shape))
    return pl.pallas_call(
        _mlp_pool_body,
        grid=(_NBLK,),
        in_specs=[
            pl.BlockSpec((_BLK, _H), lambda i: (i, 0)),
            pl.BlockSpec((_NC, _BLK, _H), lambda i: (0, i, 0)),
            pl.BlockSpec((_BLK, 1), lambda i: (i, 0)),
            full((_H, _H)), full((1, _H)), full((1, _H)), full((1, _H)),
            full((_H, _H)), full((1, _H)),
        ],
        out_specs=[
            pl.BlockSpec((_BLK, _H), lambda i: (i, 0)),
            pl.BlockSpec((_G, _H), lambda i: (0, 0)),
        ],
        out_shape=[
            jax.ShapeDtypeStruct((_N, _H), jnp.float32),
            jax.ShapeDtypeStruct((_G, _H), jnp.float32),
        ],
    )(h, acc, batch2, W1, b1, g, be, W2, b2)


def _head_body(p_ref, Wf_ref, bf_ref, Wo_ref, bo_ref, lg_ref, lp_ref):
    hf = jnp.dot(p_ref[...], Wf_ref[...], preferred_element_type=jnp.float32)
    hf = jnp.maximum(hf + bf_ref[...], 0.0)
    lg = jnp.dot(hf, Wo_ref[...], preferred_element_type=jnp.float32)
    lg = lg + bo_ref[...]
    m = jnp.max(lg, axis=1, keepdims=True)
    e = jnp.exp(lg - m)
    lse = jnp.log(jnp.sum(e, axis=1, keepdims=True)) + m
    lg_ref[...] = lg
    lp_ref[...] = lg - lse


def _tc_head(pcat, Wfin, bfin, Wo_pad, bo_pad):
    return pl.pallas_call(
        _head_body,
        out_shape=[
            jax.ShapeDtypeStruct((_G, _H), jnp.float32),
            jax.ShapeDtypeStruct((_G, _H), jnp.float32),
        ],
    )(pcat, Wfin, bfin, Wo_pad, bo_pad)


def kernel(x, edge_index, batch, num_layers,
           W1_0, b1_0, g_0, be_0, W2_0, b2_0,
           W1_1, b1_1, g_1, be_1, W2_1, b2_1,
           W1_2, b1_2, g_2, be_2, W2_2, b2_2,
           Wfin, bfin, Wout, bout):
    params = [
        (W1_0, b1_0, g_0, be_0, W2_0, b2_0),
        (W1_1, b1_1, g_1, be_1, W2_1, b2_1),
        (W1_2, b1_2, g_2, be_2, W2_2, b2_2),
    ]
    pad = _EP - _E
    src = jnp.concatenate(
        [edge_index[0], jnp.zeros((pad,), jnp.int32)]).reshape(
            _NW, _NCHUNK // _CPP, _CPP, _CH)
    dst = jnp.concatenate(
        [edge_index[1], jnp.full((pad,), _N, jnp.int32)]).reshape(
            _NW, _NCHUNK // _CPP, _CPP, _CH)
    batch2 = batch.reshape(_N, 1)
    zrows = jnp.zeros((_NP, _H), jnp.float32)

    h = x
    pooled = []
    for (W1, b1, g, be, W2, b2) in params:
        acc = _sc_segment_sum(h, src, dst, zrows)
        h, p = _tc_mlp_pool(h, acc, batch2, W1,
                            b1.reshape(1, _H), g.reshape(1, _H),
                            be.reshape(1, _H), W2, b2.reshape(1, _H))
        pooled.append(p)

    pcat = jnp.concatenate(pooled, axis=1)
    Wo_pad = jnp.pad(Wout, ((0, 0), (0, _H - _C)))
    bo_pad = jnp.concatenate(
        [bout, jnp.full((_H - _C,), -1e30, jnp.float32)]).reshape(1, _H)
    lg, lp = _tc_head(pcat, Wfin, bfin.reshape(1, _H * 3), Wo_pad, bo_pad)
    return (lg[:, :_C], lp[:, :_C])


# speedup vs baseline: 3.3565x; 3.3565x over previous
import functools

import jax
import jax.numpy as jnp
import numpy as np
from jax import lax
from jax.experimental import pallas as pl
from jax.experimental.pallas import tpu as pltpu
from jax.experimental.pallas import tpu_sc as plsc

_N = 10000
_E = 320000
_H = 128
_G = 128
_C = 10
_NC = 2
_NS = 16
_NW = _NC * _NS
_CH = 120
_NCHUNK = 84
_CPP = 42
_NRING = 2
_EPW = _NCHUNK * _CH
_EP = _NW * _EPW
_NP = 10240
_RPT = _NP // _NS
_BN = float(1.0 / np.sqrt(1.0 + 1e-5))
_BLK = 2000
_NBLK = _N // _BLK


def _sc_segment_sum(h, src, dst, zrows):
    mesh = plsc.VectorSubcoreMesh(
        core_axis_name="c", subcore_axis_name="s",
        num_cores=_NC, num_subcores=_NS)

    @functools.partial(
        pl.kernel,
        out_type=jax.ShapeDtypeStruct((_NC, _NP, _H), jnp.float32),
        mesh=mesh,
        scratch_types=[
            pltpu.VMEM((_CPP, _CH), jnp.int32),
            pltpu.VMEM((_CPP, _CH), jnp.int32),
            [pltpu.VMEM((_CH, _H), jnp.float32) for _ in range(_NRING)],
            pltpu.VMEM_SHARED((_NP, _H), jnp.float32),
            [pltpu.SemaphoreType.DMA for _ in range(_NRING)],
        ],
    )
    def seg_sum(h_hbm, src_hbm, dst_hbm, z_hbm, out_hbm,
                src_v, dst_v, rows, acc_sh, sems):
        c = lax.axis_index("c")
        s = lax.axis_index("s")
        wid = s * _NC + c
        rbase = s * _RPT

        def phase(p, carry):
            pltpu.sync_copy(src_hbm.at[wid, p], src_v)
            pltpu.sync_copy(dst_hbm.at[wid, p], dst_v)

            for b in range(_NRING):
                pltpu.async_copy(h_hbm.at[src_v.at[b]], rows[b], sems[b])

            @pl.when(p == 0)
            def _():
                pltpu.sync_copy(z_hbm.at[pl.ds(rbase, _RPT)],
                                acc_sh.at[pl.ds(rbase, _RPT)])
                plsc.subcore_barrier()

            def body(j, carry2):
                for b in range(_NRING):
                    ch = _NRING * j + b
                    pltpu.make_async_copy(h_hbm.at[src_v.at[ch]], rows[b],
                                          sems[b]).wait()
                    pltpu.sync_copy(rows[b], acc_sh.at[dst_v.at[ch]],
                                    add=True)

                    @pl.when(ch + _NRING < _CPP)
                    def _():
                        pltpu.async_copy(h_hbm.at[src_v.at[ch + _NRING]],
                                         rows[b], sems[b])
                return carry2

            lax.fori_loop(0, _CPP // _NRING, body, 0)
            return carry

        lax.fori_loop(0, _NCHUNK // _CPP, phase, 0)
        plsc.subcore_barrier()
        pltpu.sync_copy(acc_sh.at[pl.ds(rbase, _RPT)],
                        out_hbm.at[c, pl.ds(rbase, _RPT)])

    return seg_sum(h, src, dst, zrows)


def _mlp_pool_body(h_ref, acc_ref, b_ref, W1_ref, b1_ref, g_ref, be_ref,
                   W2_ref, b2_ref, hout_ref, pool_ref):
    i = pl.program_id(0)
    hin = h_ref[...] + acc_ref[0] + acc_ref[1]
    a = jnp.dot(hin, W1_ref[...], preferred_element_type=jnp.float32)
    a = (a + b1_ref[...]) * (g_ref[...] * _BN) + be_ref[...]
    a = jnp.maximum(a, 0.0)
    hout = jnp.dot(a, W2_ref[...], preferred_element_type=jnp.float32)
    hout = jnp.maximum(hout + b2_ref[...], 0.0)
    hout_ref[...] = hout
    bvals = b_ref[...][:, 0]
    onehot = (lax.broadcasted_iota(jnp.int32, (_G, _BLK), 0)
              == bvals[None, :]).astype(jnp.float32)
    pp = jnp.dot(onehot, hout, preferred_element_type=jnp.float32)

    @pl.when(i == 0)
    def _():
        pool_ref[...] = jnp.zeros_like(pool_ref)

    pool_ref[...] += pp


def _tc_mlp_pool(h, acc, batch2, W1, b1, g, be, W2, b2):
    full = lambda shape: pl.BlockSpec(shape, lambda i: (0,) * len(shape))
    return pl.pallas_call(
        _mlp_pool_body,
        grid=(_NBLK,),
        in_specs=[
            pl.BlockSpec((_BLK, _H), lambda i: (i, 0)),
            pl.BlockSpec((_NC, _BLK, _H), lambda i: (0, i, 0)),
            pl.BlockSpec((_BLK, 1), lambda i: (i, 0)),
            full((_H, _H)), full((1, _H)), full((1, _H)), full((1, _H)),
            full((_H, _H)), full((1, _H)),
        ],
        out_specs=[
            pl.BlockSpec((_BLK, _H), lambda i: (i, 0)),
            pl.BlockSpec((_G, _H), lambda i: (0, 0)),
        ],
        out_shape=[
            jax.ShapeDtypeStruct((_N, _H), jnp.float32),
            jax.ShapeDtypeStruct((_G, _H), jnp.float32),
        ],
    )(h, acc, batch2, W1, b1, g, be, W2, b2)


def _head_body(p_ref, Wf_ref, bf_ref, Wo_ref, bo_ref, lg_ref, lp_ref):
    hf = jnp.dot(p_ref[...], Wf_ref[...], preferred_element_type=jnp.float32)
    hf = jnp.maximum(hf + bf_ref[...], 0.0)
    lg = jnp.dot(hf, Wo_ref[...], preferred_element_type=jnp.float32)
    lg = lg + bo_ref[...]
    m = jnp.max(lg, axis=1, keepdims=True)
    e = jnp.exp(lg - m)
    lse = jnp.log(jnp.sum(e, axis=1, keepdims=True)) + m
    lg_ref[...] = lg
    lp_ref[...] = lg - lse


def _tc_head(pcat, Wfin, bfin, Wo_pad, bo_pad):
    return pl.pallas_call(
        _head_body,
        out_shape=[
            jax.ShapeDtypeStruct((_G, _H), jnp.float32),
            jax.ShapeDtypeStruct((_G, _H), jnp.float32),
        ],
    )(pcat, Wfin, bfin, Wo_pad, bo_pad)


def kernel(x, edge_index, batch, num_layers,
           W1_0, b1_0, g_0, be_0, W2_0, b2_0,
           W1_1, b1_1, g_1, be_1, W2_1, b2_1,
           W1_2, b1_2, g_2, be_2, W2_2, b2_2,
           Wfin, bfin, Wout, bout):
    params = [
        (W1_0, b1_0, g_0, be_0, W2_0, b2_0),
        (W1_1, b1_1, g_1, be_1, W2_1, b2_1),
        (W1_2, b1_2, g_2, be_2, W2_2, b2_2),
    ]
    pad = _EP - _E
    src = jnp.concatenate(
        [edge_index[0], jnp.zeros((pad,), jnp.int32)]).reshape(
            _NW, _NCHUNK // _CPP, _CPP, _CH)
    dst = jnp.concatenate(
        [edge_index[1], jnp.full((pad,), _N, jnp.int32)]).reshape(
            _NW, _NCHUNK // _CPP, _CPP, _CH)
    batch2 = batch.reshape(_N, 1)
    zrows = jnp.zeros((_NP, _H), jnp.float32)

    h = x
    pooled = []
    for (W1, b1, g, be, W2, b2) in params:
        acc = _sc_segment_sum(h, src, dst, zrows)
        h, p = _tc_mlp_pool(h, acc, batch2, W1,
                            b1.reshape(1, _H), g.reshape(1, _H),
                            be.reshape(1, _H), W2, b2.reshape(1, _H))
        pooled.append(p)

    pcat = jnp.concatenate(pooled, axis=1)
    Wo_pad = jnp.pad(Wout, ((0, 0), (0, _H - _C)))
    bo_pad = jnp.concatenate(
        [bout, jnp.full((_H - _C,), -1e30, jnp.float32)]).reshape(1, _H)
    lg, lp = _tc_head(pcat, Wfin, bfin.reshape(1, _H * 3), Wo_pad, bo_pad)
    return (lg[:, :_C], lp[:, :_C])
